# baseline (speedup 1.0000x reference)
import functools

import jax
import jax.numpy as jnp
from jax import lax
from jax.experimental import pallas as pl
from jax.experimental.pallas import tpu as pltpu
from jax.experimental.pallas import tpu_sc as plsc

N = 16 * 512 * 512
NC = 2
NS = 16
L = 16
NW = NC * NS
PER_W = N // NW
CHUNK = 16384
NCHUNK = PER_W // CHUNK

_mesh = plsc.VectorSubcoreMesh(core_axis_name="c", subcore_axis_name="s")


@functools.partial(
    pl.kernel,
    mesh=_mesh,
    out_type=jax.ShapeDtypeStruct((NW, 4, L), jnp.float32),
    scratch_types=[
        pltpu.VMEM((CHUNK,), jnp.float32),
        pltpu.VMEM((CHUNK,), jnp.float32),
        pltpu.VMEM((4, L), jnp.float32),
    ],
)
def _flood_mse_partials(a_hbm, b_hbm, out_hbm, a_v, b_v, res_v):
    wid = lax.axis_index("s") * NC + lax.axis_index("c")
    base = wid * PER_W
    zero = jnp.zeros((L,), jnp.float32)

    def chunk_body(c, accs):
        pltpu.sync_copy(a_hbm.at[pl.ds(base + c * CHUNK, CHUNK)], a_v)
        pltpu.sync_copy(b_hbm.at[pl.ds(base + c * CHUNK, CHUNK)], b_v)

        def inner(i, accs2):
            sl, cl, sp, cp = accs2
            a = a_v[pl.ds(i * L, L)]
            b = b_v[pl.ds(i * L, L)]
            d = a - b
            sq = d * d
            ml = b > 0.0
            mp = a > 0.0
            sl = sl + jnp.where(ml, sq, 0.0)
            cl = cl + jnp.where(ml, 1.0, 0.0)
            sp = sp + jnp.where(mp, sq, 0.0)
            cp = cp + jnp.where(mp, 1.0, 0.0)
            return (sl, cl, sp, cp)

        return lax.fori_loop(0, CHUNK // L, inner, accs)

    sl, cl, sp, cp = lax.fori_loop(
        0, NCHUNK, chunk_body, (zero, zero, zero, zero)
    )
    res_v[0, :] = sl
    res_v[1, :] = cl
    res_v[2, :] = sp
    res_v[3, :] = cp
    pltpu.sync_copy(res_v, out_hbm.at[wid])


def kernel(inputs, targets):
    a = inputs.reshape(-1)
    b = targets.reshape(-1)
    parts = _flood_mse_partials(a, b)
    sums = jnp.sum(parts, axis=(0, 2))
    loss_label = sums[0] / sums[1]
    loss_pred = sums[2] / sums[3]
    return (loss_label + loss_pred, loss_label, loss_pred)


# speedup vs baseline: 1.0526x; 1.0526x over previous
import functools

import jax
import jax.numpy as jnp
from jax import lax
from jax.experimental import pallas as pl
from jax.experimental.pallas import tpu as pltpu
from jax.experimental.pallas import tpu_sc as plsc

N = 16 * 512 * 512
NC = 2
NS = 16
L = 16
NW = NC * NS
PER_W = N // NW
CHUNK = 16384
NCHUNK = PER_W // CHUNK

_mesh = plsc.VectorSubcoreMesh(core_axis_name="c", subcore_axis_name="s")


@functools.partial(
    pl.kernel,
    mesh=_mesh,
    out_type=jax.ShapeDtypeStruct((NW, 4, L), jnp.float32),
    scratch_types=[
        pltpu.VMEM((2, CHUNK), jnp.float32),
        pltpu.VMEM((2, CHUNK), jnp.float32),
        pltpu.VMEM((4, L), jnp.float32),
        pltpu.SemaphoreType.DMA,
        pltpu.SemaphoreType.DMA,
        pltpu.SemaphoreType.DMA,
        pltpu.SemaphoreType.DMA,
    ],
)
def _flood_mse_partials(a_hbm, b_hbm, out_hbm, a_v, b_v, res_v,
                        sa0, sa1, sb0, sb1):
    wid = lax.axis_index("s") * NC + lax.axis_index("c")
    base = wid * PER_W
    sems_a = (sa0, sa1)
    sems_b = (sb0, sb1)

    def dma_pair(c, slot):
        off = base + c * CHUNK
        ca = pltpu.make_async_copy(
            a_hbm.at[pl.ds(off, CHUNK)], a_v.at[slot], sems_a[slot])
        cb = pltpu.make_async_copy(
            b_hbm.at[pl.ds(off, CHUNK)], b_v.at[slot], sems_b[slot])
        return ca, cb

    for c in (0, 1):
        ca, cb = dma_pair(c, c)
        ca.start()
        cb.start()

    zf = jnp.zeros((L,), jnp.float32)
    zi = jnp.zeros((L,), jnp.int32)
    accs = (zf, zi, zf, zi)

    for c in range(NCHUNK):
        slot = c % 2
        ca, cb = dma_pair(c, slot)
        ca.wait()
        cb.wait()
        av = a_v.at[slot]
        bv = b_v.at[slot]

        @plsc.parallel_loop(0, CHUNK, L, unroll=8, carry=accs)
        def accs(i, accs2):
            sl, cl, sp, cp = accs2
            a = av[pl.ds(i, L)]
            b = bv[pl.ds(i, L)]
            d = a - b
            sq = d * d
            ml = b > 0.0
            mp = a > 0.0
            sl = sl + jnp.where(ml, sq, 0.0)
            sp = sp + jnp.where(mp, sq, 0.0)
            cl = cl + jnp.where(ml, 1, 0)
            cp = cp + jnp.where(mp, 1, 0)
            return (sl, cl, sp, cp)

        if c + 2 < NCHUNK:
            na, nb = dma_pair(c + 2, slot)
            na.start()
            nb.start()

    sl, cl, sp, cp = accs
    res_v[0, :] = sl
    res_v[1, :] = cl.astype(jnp.float32)
    res_v[2, :] = sp
    res_v[3, :] = cp.astype(jnp.float32)
    pltpu.sync_copy(res_v, out_hbm.at[wid])


def kernel(inputs, targets):
    a = inputs.reshape(-1)
    b = targets.reshape(-1)
    parts = _flood_mse_partials(a, b)
    sums = jnp.sum(parts, axis=(0, 2))
    loss_label = sums[0] / sums[1]
    loss_pred = sums[2] / sums[3]
    return (loss_label + loss_pred, loss_label, loss_pred)


# speedup vs baseline: 2.1966x; 2.0869x over previous
import functools

import jax
import jax.numpy as jnp
from jax import lax
from jax.experimental import pallas as pl
from jax.experimental.pallas import tpu as pltpu
from jax.experimental.pallas import tpu_sc as plsc

NIMG = 16
ROWS = 512
COLS = 512
NC = 2
NS = 16
L = 16
NW = NC * NS
ROWS_W = NIMG * ROWS // NW
CHUNKR = 32
NCHUNK = ROWS_W // CHUNKR
BANKS = 4
STEP = BANKS * L

_mesh = plsc.VectorSubcoreMesh(core_axis_name="c", subcore_axis_name="s")


@functools.partial(
    pl.kernel,
    mesh=_mesh,
    out_type=jax.ShapeDtypeStruct((NW, 4, L), jnp.float32),
    scratch_types=[
        pltpu.VMEM((2, CHUNKR, COLS), jnp.float32),
        pltpu.VMEM((2, CHUNKR, COLS), jnp.float32),
        pltpu.VMEM((4, L), jnp.float32),
        pltpu.SemaphoreType.DMA,
        pltpu.SemaphoreType.DMA,
        pltpu.SemaphoreType.DMA,
        pltpu.SemaphoreType.DMA,
    ],
)
def _flood_mse_partials(a_hbm, b_hbm, out_hbm, a_v, b_v, res_v,
                        sa0, sa1, sb0, sb1):
    wid = lax.axis_index("s") * NC + lax.axis_index("c")
    img = wid // 2
    row0 = (wid % 2) * ROWS_W
    sems_a = (sa0, sa1)
    sems_b = (sb0, sb1)

    def dma_pair(c, slot):
        r = row0 + c * CHUNKR
        ca = pltpu.make_async_copy(
            a_hbm.at[img, 0, pl.ds(r, CHUNKR), :], a_v.at[slot],
            sems_a[slot])
        cb = pltpu.make_async_copy(
            b_hbm.at[img, 0, pl.ds(r, CHUNKR), :], b_v.at[slot],
            sems_b[slot])
        return ca, cb

    for c in (0, 1):
        ca, cb = dma_pair(c, c)
        ca.start()
        cb.start()

    zf = jnp.zeros((L,), jnp.float32)
    zi = jnp.zeros((L,), jnp.int32)
    accs = tuple((zf, zi, zf, zi) for _ in range(BANKS))

    for c in range(NCHUNK):
        slot = c % 2
        ca, cb = dma_pair(c, slot)
        ca.wait()
        cb.wait()
        av = a_v.at[slot]
        bv = b_v.at[slot]

        @plsc.parallel_loop(0, CHUNKR * COLS, STEP, unroll=2, carry=accs)
        def accs(i, banks):
            r = jax.lax.shift_right_logical(i, 9)
            col = pl.multiple_of(jax.lax.bitwise_and(i, COLS - 1), STEP)
            out = []
            for j in range(BANKS):
                sl, cl, sp, cp = banks[j]
                a = av[r, pl.ds(col + j * L, L)]
                b = bv[r, pl.ds(col + j * L, L)]
                d = a - b
                sq = d * d
                ml = b > 0.0
                mp = a > 0.0
                sl = sl + jnp.where(ml, sq, 0.0)
                sp = sp + jnp.where(mp, sq, 0.0)
                cl = cl + jnp.where(ml, 1, 0)
                cp = cp + jnp.where(mp, 1, 0)
                out.append((sl, cl, sp, cp))
            return tuple(out)

        if c + 2 < NCHUNK:
            na, nb = dma_pair(c + 2, slot)
            na.start()
            nb.start()

    sl = accs[0][0]
    cl = accs[0][1]
    sp = accs[0][2]
    cp = accs[0][3]
    for j in range(1, BANKS):
        sl = sl + accs[j][0]
        cl = cl + accs[j][1]
        sp = sp + accs[j][2]
        cp = cp + accs[j][3]
    res_v[0, :] = sl
    res_v[1, :] = cl.astype(jnp.float32)
    res_v[2, :] = sp
    res_v[3, :] = cp.astype(jnp.float32)
    pltpu.sync_copy(res_v, out_hbm.at[wid])


def kernel(inputs, targets):
    parts = _flood_mse_partials(inputs, targets)
    sums = jnp.sum(parts, axis=(0, 2))
    loss_label = sums[0] / sums[1]
    loss_pred = sums[2] / sums[3]
    return (loss_label + loss_pred, loss_label, loss_pred)


# speedup vs baseline: 2.5618x; 1.1662x over previous
import functools

import jax
import jax.numpy as jnp
from jax import lax
from jax.experimental import pallas as pl
from jax.experimental.pallas import tpu as pltpu
from jax.experimental.pallas import tpu_sc as plsc

NIMG = 16
ROWS = 512
COLS = 512
K_SC = 8
K_TC = NIMG - K_SC
NC = 2
NS = 16
L = 16
NW = NC * NS
ROWS_W = K_SC * ROWS // NW
CHUNKR = 32
NCHUNK = ROWS_W // CHUNKR
BANKS = 4
STEP = BANKS * L

_mesh = plsc.VectorSubcoreMesh(core_axis_name="c", subcore_axis_name="s")


@functools.partial(
    pl.kernel,
    mesh=_mesh,
    out_type=jax.ShapeDtypeStruct((NW, 4, L), jnp.float32),
    scratch_types=[
        pltpu.VMEM((2, CHUNKR, COLS), jnp.float32),
        pltpu.VMEM((2, CHUNKR, COLS), jnp.float32),
        pltpu.VMEM((4, L), jnp.float32),
        pltpu.SemaphoreType.DMA,
        pltpu.SemaphoreType.DMA,
        pltpu.SemaphoreType.DMA,
        pltpu.SemaphoreType.DMA,
    ],
)
def _flood_mse_sc(a_hbm, b_hbm, out_hbm, a_v, b_v, res_v,
                  sa0, sa1, sb0, sb1):
    wid = lax.axis_index("s") * NC + lax.axis_index("c")
    grow0 = wid * ROWS_W
    sems_a = (sa0, sa1)
    sems_b = (sb0, sb1)

    def dma_pair(c, slot):
        gr = grow0 + c * CHUNKR
        img = gr // ROWS
        r = gr % ROWS
        ca = pltpu.make_async_copy(
            a_hbm.at[img, 0, pl.ds(r, CHUNKR), :], a_v.at[slot],
            sems_a[slot])
        cb = pltpu.make_async_copy(
            b_hbm.at[img, 0, pl.ds(r, CHUNKR), :], b_v.at[slot],
            sems_b[slot])
        return ca, cb

    for c in (0, 1):
        ca, cb = dma_pair(c, c)
        ca.start()
        cb.start()

    zf = jnp.zeros((L,), jnp.float32)
    zi = jnp.zeros((L,), jnp.int32)
    accs = tuple((zf, zi, zf, zi) for _ in range(BANKS))

    for c in range(NCHUNK):
        slot = c % 2
        ca, cb = dma_pair(c, slot)
        ca.wait()
        cb.wait()
        av = a_v.at[slot]
        bv = b_v.at[slot]

        @plsc.parallel_loop(0, CHUNKR * COLS, STEP, unroll=2, carry=accs)
        def accs(i, banks):
            r = jax.lax.shift_right_logical(i, 9)
            col = pl.multiple_of(jax.lax.bitwise_and(i, COLS - 1), STEP)
            out = []
            for j in range(BANKS):
                sl, cl, sp, cp = banks[j]
                a = av[r, pl.ds(col + j * L, L)]
                b = bv[r, pl.ds(col + j * L, L)]
                d = a - b
                sq = d * d
                ml = b > 0.0
                mp = a > 0.0
                sl = sl + jnp.where(ml, sq, 0.0)
                sp = sp + jnp.where(mp, sq, 0.0)
                cl = cl + jnp.where(ml, 1, 0)
                cp = cp + jnp.where(mp, 1, 0)
                out.append((sl, cl, sp, cp))
            return tuple(out)

        if c + 2 < NCHUNK:
            na, nb = dma_pair(c + 2, slot)
            na.start()
            nb.start()

    sl, cl, sp, cp = accs[0]
    for j in range(1, BANKS):
        sl = sl + accs[j][0]
        cl = cl + accs[j][1]
        sp = sp + accs[j][2]
        cp = cp + accs[j][3]
    res_v[0, :] = sl
    res_v[1, :] = cl.astype(jnp.float32)
    res_v[2, :] = sp
    res_v[3, :] = cp.astype(jnp.float32)
    pltpu.sync_copy(res_v, out_hbm.at[wid])


def _flood_mse_tc_body(a_ref, b_ref, o_ref):
    i = pl.program_id(0)
    a = a_ref[0, 0]
    b = b_ref[0, 0]
    d = a - b
    sq = d * d
    ml = b > 0.0
    mp = a > 0.0
    sl = jnp.sum(jnp.where(ml, sq, 0.0))
    cl = jnp.sum(jnp.where(ml, 1.0, 0.0))
    sp = jnp.sum(jnp.where(mp, sq, 0.0))
    cp = jnp.sum(jnp.where(mp, 1.0, 0.0))

    @pl.when(i == 0)
    def _():
        o_ref[0] = 0.0
        o_ref[1] = 0.0
        o_ref[2] = 0.0
        o_ref[3] = 0.0

    o_ref[0] += sl
    o_ref[1] += cl
    o_ref[2] += sp
    o_ref[3] += cp


_flood_mse_tc = pl.pallas_call(
    _flood_mse_tc_body,
    grid=(K_TC,),
    in_specs=[
        pl.BlockSpec((1, 1, ROWS, COLS), lambda i: (K_SC + i, 0, 0, 0)),
        pl.BlockSpec((1, 1, ROWS, COLS), lambda i: (K_SC + i, 0, 0, 0)),
    ],
    out_specs=pl.BlockSpec(memory_space=pltpu.SMEM),
    out_shape=jax.ShapeDtypeStruct((4,), jnp.float32),
)


def kernel(inputs, targets):
    parts_sc = _flood_mse_sc(inputs, targets)
    parts_tc = _flood_mse_tc(inputs, targets)
    sums = jnp.sum(parts_sc, axis=(0, 2)) + parts_tc
    loss_label = sums[0] / sums[1]
    loss_pred = sums[2] / sums[3]
    return (loss_label + loss_pred, loss_label, loss_pred)
